# baseline (speedup 1.0000x reference)
import functools

import jax
import jax.numpy as jnp
from jax import lax
from jax.experimental import pallas as pl
from jax.experimental.pallas import tpu as pltpu
from jax.experimental.pallas import tpu_sc as plsc

NUM_CORES = 2
NUM_SUBCORES = 16
NUM_WORKERS = NUM_CORES * NUM_SUBCORES
CHUNK_A = 64
CHUNK_B = 32


def kernel(tokens, embed_table):
    batch = tokens.shape[0]
    seq_len = tokens.shape[1]
    embed_dim = embed_table.shape[1]
    rows_per_worker = seq_len // NUM_WORKERS
    chunks = []
    off = 0
    while off < rows_per_worker:
        size = CHUNK_A if len(chunks) % 2 == 0 else CHUNK_B
        size = min(size, rows_per_worker - off)
        chunks.append((off, size))
        off += size
    n_chunks = len(chunks)
    mesh = plsc.VectorSubcoreMesh(core_axis_name="c", subcore_axis_name="s")

    @functools.partial(
        pl.kernel,
        mesh=mesh,
        out_type=jax.ShapeDtypeStruct(
            (batch, seq_len, embed_dim), embed_table.dtype),
        scratch_types=[
            pltpu.VMEM((CHUNK_A, embed_dim), jnp.float32),
            pltpu.VMEM((CHUNK_B, embed_dim), jnp.float32),
            pltpu.SemaphoreType.DMA,
            pltpu.SemaphoreType.DMA,
            pltpu.SemaphoreType.DMA,
            pltpu.SemaphoreType.DMA,
        ],
    )
    def sc_copy(table_hbm, out_hbm, buf_a, buf_b, rsem_a, rsem_b,
                wsem_a, wsem_b):
        wid = lax.axis_index("s") * NUM_CORES + lax.axis_index("c")
        base = wid * rows_per_worker
        bufs = (buf_a, buf_b)
        rsems = (rsem_a, rsem_b)
        wsems = (wsem_a, wsem_b)

        def read(i):
            off, size = chunks[i]
            s = i % 2
            return pltpu.async_copy(
                table_hbm.at[pl.ds(base + off, size)],
                bufs[s].at[pl.ds(0, size)], rsems[s])

        def writes(i):
            off, size = chunks[i]
            s = i % 2
            return [
                pltpu.async_copy(
                    bufs[s].at[pl.ds(0, size)],
                    out_hbm.at[b, pl.ds(base + off, size)], wsems[s])
                for b in range(batch)
            ]

        pending_reads = {0: read(0)}
        pending_writes = {}
        for i in range(n_chunks):
            pending_reads.pop(i).wait()
            pending_writes[i] = writes(i)
            if i + 1 < n_chunks:
                if i >= 1:
                    for h in pending_writes.pop(i - 1):
                        h.wait()
                pending_reads[i + 1] = read(i + 1)
        for i in sorted(pending_writes):
            for h in pending_writes.pop(i):
                h.wait()

    return sc_copy(embed_table[:seq_len])
